# Initial kernel scaffold; baseline (speedup 1.0000x reference)
import jax
import jax.numpy as jnp
from jax.experimental import pallas as pl


def kernel(x, expert_indices, w1, w2, w3):
    raise NotImplementedError("write your pallas kernel here")


# speedup vs baseline: 8.0186x; 8.0186x over previous
import jax
import jax.numpy as jnp
from jax.experimental import pallas as pl
from jax.experimental.pallas import tpu as pltpu

_IB = 256


def _ffn_kernel(idx_ref, x_ref, w1_ref, w3_ref, w2_ref, out_ref, acc_ref):
    e = pl.program_id(0)
    j = pl.program_id(1)
    nj = pl.num_programs(1)

    @pl.when((e == 0) & (j == 0))
    def _():
        out_ref[...] = jnp.zeros_like(out_ref)

    xv = x_ref[...]
    g = jax.lax.dot_general(xv, w1_ref[...], (((1,), (1,)), ((), ())),
                            preferred_element_type=jnp.float32)
    u = jax.lax.dot_general(xv, w3_ref[...], (((1,), (1,)), ((), ())),
                            preferred_element_type=jnp.float32)
    h = g * jax.nn.sigmoid(g) * u
    pe = jax.lax.dot_general(h, w2_ref[...], (((1,), (1,)), ((), ())),
                             preferred_element_type=jnp.float32)

    @pl.when(j == 0)
    def _():
        acc_ref[...] = pe

    @pl.when(j > 0)
    def _():
        acc_ref[...] += pe

    @pl.when(j == nj - 1)
    def _():
        m = (idx_ref[...] == e).astype(jnp.float32)
        out_ref[...] += m[:, :, None] * acc_ref[...][None, :, :]


def kernel(x, expert_indices, w1, w2, w3):
    E, I, D = w1.shape
    T = x.shape[0]
    A = expert_indices.shape[1]
    idx = expert_indices.astype(jnp.int32).T

    nj = I // _IB
    out = pl.pallas_call(
        _ffn_kernel,
        grid=(E, nj),
        in_specs=[
            pl.BlockSpec((A, T), lambda e, j: (0, 0)),
            pl.BlockSpec((T, D), lambda e, j: (0, 0)),
            pl.BlockSpec((None, _IB, D), lambda e, j: (e, j, 0)),
            pl.BlockSpec((None, _IB, D), lambda e, j: (e, j, 0)),
            pl.BlockSpec((None, D, _IB), lambda e, j: (e, 0, j)),
        ],
        out_specs=pl.BlockSpec((A, T, D), lambda e, j: (0, 0, 0)),
        out_shape=jax.ShapeDtypeStruct((A, T, D), jnp.float32),
        scratch_shapes=[pltpu.VMEM((T, D), jnp.float32)],
        compiler_params=pltpu.CompilerParams(
            dimension_semantics=("arbitrary", "arbitrary"),
        ),
    )(idx, x, w1, w3, w2)
    return out.transpose(1, 0, 2)


# speedup vs baseline: 11.4218x; 1.4244x over previous
import jax
import jax.numpy as jnp
from jax.experimental import pallas as pl
from jax.experimental.pallas import tpu as pltpu

_IB = 1408


def _ffn_kernel(idx_ref, x_ref, w1_ref, w3_ref, w2_ref, out_ref, acc_ref):
    e = pl.program_id(0)
    j = pl.program_id(1)
    nj = pl.num_programs(1)

    @pl.when((e == 0) & (j == 0))
    def _():
        out_ref[...] = jnp.zeros_like(out_ref)

    xv = x_ref[...]
    g = jax.lax.dot_general(xv, w1_ref[...], (((1,), (1,)), ((), ())),
                            preferred_element_type=jnp.float32)
    u = jax.lax.dot_general(xv, w3_ref[...], (((1,), (1,)), ((), ())),
                            preferred_element_type=jnp.float32)
    h = g * jax.nn.sigmoid(g) * u
    pe = jax.lax.dot_general(h, w2_ref[...], (((1,), (1,)), ((), ())),
                             preferred_element_type=jnp.float32)

    @pl.when(j == 0)
    def _():
        acc_ref[...] = pe

    @pl.when(j > 0)
    def _():
        acc_ref[...] += pe

    @pl.when(j == nj - 1)
    def _():
        m = (idx_ref[...] == e).astype(jnp.float32)
        out_ref[...] += m[:, :, None] * acc_ref[...][None, :, :]


def kernel(x, expert_indices, w1, w2, w3):
    E, I, D = w1.shape
    T = x.shape[0]
    A = expert_indices.shape[1]
    idx = expert_indices.astype(jnp.int32).T

    nj = I // _IB
    out = pl.pallas_call(
        _ffn_kernel,
        grid=(E, nj),
        in_specs=[
            pl.BlockSpec((A, T), lambda e, j: (0, 0)),
            pl.BlockSpec((T, D), lambda e, j: (0, 0)),
            pl.BlockSpec((None, _IB, D), lambda e, j: (e, j, 0)),
            pl.BlockSpec((None, _IB, D), lambda e, j: (e, j, 0)),
            pl.BlockSpec((None, D, _IB), lambda e, j: (e, 0, j)),
        ],
        out_specs=pl.BlockSpec((A, T, D), lambda e, j: (0, 0, 0)),
        out_shape=jax.ShapeDtypeStruct((A, T, D), jnp.float32),
        scratch_shapes=[pltpu.VMEM((T, D), jnp.float32)],
        compiler_params=pltpu.CompilerParams(
            dimension_semantics=("arbitrary", "arbitrary"),
        ),
    )(idx, x, w1, w3, w2)
    return out.transpose(1, 0, 2)
